# baseline (speedup 1.0000x reference)
import functools

import jax
import jax.numpy as jnp
from jax.experimental import pallas as pl
from jax.experimental.pallas import tpu as pltpu
from jax.experimental.pallas import tpu_sc as plsc

_BM = 256


def _router_body(hs_ref, gw_ref, logits_ref, pos_ref, w_ref, sg_ref, bex_ref):
    hs = hs_ref[...]
    T = hs.shape[0]
    EP = gw_ref.shape[1]
    E = EP - 1
    NBm = (2 * T) // _BM + E
    NB = NBm + T // _BM
    l9 = jnp.dot(hs, gw_ref[...], preferred_element_type=jnp.float32)
    logits_ref[...] = l9[:, :E]
    lane = jax.lax.broadcasted_iota(jnp.int32, (T, EP), 1)
    moe = lane < E
    lm = jnp.where(moe, l9, -1e30)
    mx = jnp.max(lm, axis=1, keepdims=True)
    ex = jnp.where(moe, jnp.exp(lm - mx), 0.0)
    rw = ex / jnp.sum(ex, axis=1, keepdims=True)
    m1 = jnp.max(rw, axis=1, keepdims=True)
    e0 = jnp.min(jnp.where(rw == m1, lane, EP), axis=1, keepdims=True)
    rw2 = jnp.where(lane == e0, -1.0, rw)
    m2 = jnp.max(rw2, axis=1, keepdims=True)
    e1 = jnp.min(jnp.where(rw2 == m2, lane, EP), axis=1, keepdims=True)
    den = m1 + m2
    w_ref[:, 0:1] = m1 / den
    w_ref[:, 1:2] = m2 / den
    sg_ref[...] = jax.nn.sigmoid(l9[:, E:EP])

    oh0 = (lane[:, :E] == e0).astype(jnp.float32)
    oh1 = (lane[:, :E] == e1).astype(jnp.float32)
    m8 = oh0 + oh1
    r_i = jax.lax.broadcasted_iota(jnp.int32, (T, T), 0)
    c_i = jax.lax.broadcasted_iota(jnp.int32, (T, T), 1)
    ltri = (c_i < r_i).astype(jnp.bfloat16)
    rank = jnp.dot(ltri, m8.astype(jnp.bfloat16),
                   preferred_element_type=jnp.float32)
    ones_row = jnp.ones((1, T), jnp.bfloat16)
    cnt = jnp.dot(ones_row, m8.astype(jnp.bfloat16),
                  preferred_element_type=jnp.float32)
    ci = cnt.astype(jnp.int32)
    pci = ((ci + (_BM - 1)) // _BM) * _BM
    pcf = pci.astype(jnp.float32)
    u_r = jax.lax.broadcasted_iota(jnp.int32, (E, E), 0)
    u_c = jax.lax.broadcasted_iota(jnp.int32, (E, E), 1)
    utri = (u_r < u_c).astype(jnp.float32)
    poff = jnp.dot(pcf, utri, preferred_element_type=jnp.float32)
    pos0 = jnp.sum((rank + poff) * oh0, axis=1, keepdims=True)
    pos1 = jnp.sum((rank + poff) * oh1, axis=1, keepdims=True)
    pos_ref[:, 0:1] = pos0.astype(jnp.int32)
    pos_ref[:, 1:2] = pos1.astype(jnp.int32)

    bio = jax.lax.broadcasted_iota(jnp.int32, (NB, 1), 0)
    bstart = (bio * _BM).astype(jnp.float32)
    ends = poff + pcf
    be8 = jnp.sum((bstart >= ends).astype(jnp.int32), axis=1, keepdims=True)
    bex_ref[...] = jnp.where(bio >= NBm, E, jnp.minimum(be8, E - 1))


def _make_sc1(T, S_moe):
    mesh = plsc.VectorSubcoreMesh(core_axis_name="c", subcore_axis_name="s")

    @functools.partial(
        pl.kernel, mesh=mesh,
        compiler_params=pltpu.CompilerParams(needs_layout_passes=False),
        out_type=[jax.ShapeDtypeStruct((S_moe,), jnp.int32),
                  jax.ShapeDtypeStruct((S_moe,), jnp.float32)],
        scratch_types=[pltpu.VMEM((2 * T,), jnp.int32),
                       pltpu.VMEM((2 * T,), jnp.float32),
                       pltpu.VMEM((S_moe,), jnp.int32),
                       pltpu.VMEM((S_moe,), jnp.float32)])
    def sc1(pos_hbm, w_hbm, tok_out, w_out, pos_v, w_v, stok_v, sw_v):
        cid = jax.lax.axis_index("c")
        sid = jax.lax.axis_index("s")

        @pl.when(jnp.logical_and(cid == 0, sid == 0))
        def _():
            pltpu.sync_copy(pos_hbm, pos_v)
            pltpu.sync_copy(w_hbm, w_v)
            zi = jnp.zeros((16,), jnp.int32)
            zf = jnp.zeros((16,), jnp.float32)

            def zbody(i, carry):
                stok_v[pl.ds(i * 16, 16)] = zi
                sw_v[pl.ds(i * 16, 16)] = zf
                return carry

            jax.lax.fori_loop(0, S_moe // 16, zbody, 0)
            lanes = jax.lax.broadcasted_iota(jnp.int32, (16,), 0)

            def sbody(i, carry):
                base = i * 16
                idx = pos_v[pl.ds(base, 16)]
                tok = jax.lax.shift_right_logical(base + lanes, 1)
                plsc.store_scatter(stok_v, [idx], tok)
                wv = w_v[pl.ds(base, 16)]
                plsc.store_scatter(sw_v, [idx], wv)
                return carry

            jax.lax.fori_loop(0, (2 * T) // 16, sbody, 0)
            pltpu.sync_copy(stok_v, tok_out)
            pltpu.sync_copy(sw_v, w_out)

    return sc1


def _make_sc2(S_moe, H):
    NW = 32
    per = S_moe // NW
    CH = 16
    NCH = per // CH
    mesh = plsc.VectorSubcoreMesh(core_axis_name="c", subcore_axis_name="s")

    @functools.partial(
        pl.kernel, mesh=mesh,
        compiler_params=pltpu.CompilerParams(needs_layout_passes=False),
        out_type=jax.ShapeDtypeStruct((S_moe, H), jnp.float32),
        scratch_types=[pltpu.VMEM((per,), jnp.int32),
                       pltpu.VMEM((CH, H), jnp.float32),
                       pltpu.VMEM((CH, H), jnp.float32),
                       pltpu.SemaphoreType.DMA,
                       pltpu.SemaphoreType.DMA,
                       pltpu.SemaphoreType.DMA,
                       pltpu.SemaphoreType.DMA])
    def sc2(tok_hbm, hs_hbm, x_out, idx_v, b0, b1, g0, g1, w0s, w1s):
        cid = jax.lax.axis_index("c")
        sid = jax.lax.axis_index("s")
        wid = sid * 2 + cid
        base = wid * per
        pltpu.sync_copy(tok_hbm.at[pl.ds(base, per)], idx_v)
        bufs = [b0, b1]
        gsems = [g0, g1]
        wsems = [w0s, w1s]
        gd = [None, None]
        wd = [None, None]
        gd[0] = pltpu.async_copy(hs_hbm.at[idx_v.at[pl.ds(0, CH)]], bufs[0],
                                 gsems[0])
        for c in range(NCH):
            cb = c % 2
            nb = (c + 1) % 2
            if c + 1 < NCH:
                if c >= 1:
                    wd[nb].wait()
                gd[nb] = pltpu.async_copy(
                    hs_hbm.at[idx_v.at[pl.ds((c + 1) * CH, CH)]], bufs[nb],
                    gsems[nb])
            gd[cb].wait()
            wd[cb] = pltpu.async_copy(bufs[cb],
                                      x_out.at[pl.ds(base + c * CH, CH)],
                                      wsems[cb])
        wd[0].wait()
        wd[1].wait()

    return sc2


def _make_sc3(T, H, S_moe):
    NW = 32
    ntok = T // NW
    CT = 16
    NCH = ntok // CT
    mesh = plsc.VectorSubcoreMesh(core_axis_name="c", subcore_axis_name="s")

    @functools.partial(
        pl.kernel, mesh=mesh,
        compiler_params=pltpu.CompilerParams(needs_layout_passes=False),
        out_type=jax.ShapeDtypeStruct((T, H), jnp.float32),
        scratch_types=[pltpu.VMEM((2 * ntok,), jnp.int32),
                       pltpu.VMEM((CT, H), jnp.float32),
                       pltpu.VMEM((CT, H), jnp.float32),
                       pltpu.VMEM((CT, H), jnp.float32),
                       pltpu.VMEM((16,), jnp.int32),
                       pltpu.VMEM((16,), jnp.int32),
                       pltpu.SemaphoreType.DMA,
                       pltpu.SemaphoreType.DMA,
                       pltpu.SemaphoreType.DMA,
                       pltpu.SemaphoreType.DMA])
    def sc3(osort_hbm, pos_hbm, fin_out, pv, b0, b1, b2, i0v, i1v,
            s0, s1, s2, ws):
        cid = jax.lax.axis_index("c")
        sid = jax.lax.axis_index("s")
        wid = sid * 2 + cid
        base_t = wid * ntok
        pltpu.sync_copy(pos_hbm.at[pl.ds(2 * base_t, 2 * ntok)], pv)
        lanes = jax.lax.broadcasted_iota(jnp.int32, (16,), 0)
        for c in range(NCH):
            toff = c * CT
            g = 2 * toff + 2 * lanes
            i0v[...] = plsc.load_gather(pv, [g])
            i1v[...] = plsc.load_gather(pv, [g + 1])
            d0 = pltpu.async_copy(osort_hbm.at[i0v], b0, s0)
            d1 = pltpu.async_copy(osort_hbm.at[i1v], b1, s1)
            d2 = pltpu.async_copy(
                osort_hbm.at[pl.ds(S_moe + base_t + toff, CT)], b2, s2)
            d0.wait()
            d1.wait()
            d2.wait()
            for r in range(CT):
                def abody(j, carry):
                    for q in range(4):
                        sl = pl.ds(j * 64 + q * 16, 16)
                        b2[r, sl] = b0[r, sl] + b1[r, sl] + b2[r, sl]
                    return carry

                jax.lax.fori_loop(0, H // 64, abody, 0)
            dw = pltpu.async_copy(b2, fin_out.at[pl.ds(base_t + toff, CT)], ws)
            dw.wait()

    return sc3


def kernel(hidden_states, gate_w, W0, b0, W1, b1, Wo, bo, sW0, sb0, sW1, sb1,
           sWo, sbo, sg_w):
    b_, s_, h_ = hidden_states.shape
    T = b_ * s_
    E = gate_w.shape[1]
    I = W0.shape[2]
    EP = E + 1
    NBm = (2 * T) // _BM + E
    NBs = T // _BM
    NB = NBm + NBs
    S_moe = NBm * _BM
    S_pad = NB * _BM
    hs2 = hidden_states.reshape(T, h_)
    gwcat = jnp.concatenate([gate_w, sg_w], axis=1)

    logits, pos, w2, sgate, bex = pl.pallas_call(
        _router_body,
        out_shape=[
            jax.ShapeDtypeStruct((T, E), jnp.float32),
            jax.ShapeDtypeStruct((T, 2), jnp.int32),
            jax.ShapeDtypeStruct((T, 2), jnp.float32),
            jax.ShapeDtypeStruct((T, 1), jnp.float32),
            jax.ShapeDtypeStruct((NB, 1), jnp.int32),
        ],
    )(hs2, gwcat)

    slot_tok, slot_w_moe = _make_sc1(T, S_moe)(pos.reshape(2 * T),
                                               w2.reshape(2 * T))
    x_sorted = _make_sc2(S_moe, h_)(slot_tok, hs2)

    bf = jnp.bfloat16
    W0c = jnp.concatenate([W0, sW0[None]], axis=0).astype(bf)
    W1c = jnp.concatenate([W1, sW1[None]], axis=0).astype(bf)
    Woc = jnp.concatenate([Wo, sWo[None]], axis=0).astype(bf)
    b0c = jnp.concatenate([b0, sb0[None]], axis=0).reshape(EP, 1, I)
    b1c = jnp.concatenate([b1, sb1[None]], axis=0).reshape(EP, 1, I)
    boc = jnp.concatenate([bo, sbo[None]], axis=0).reshape(EP, 1, h_)
    slot_w = jnp.concatenate([slot_w_moe, sgate.reshape(T)]).reshape(S_pad, 1)
    bex1 = bex.reshape(NB)

    grid_spec = pltpu.PrefetchScalarGridSpec(
        num_scalar_prefetch=1,
        grid=(NB,),
        in_specs=[
            pl.BlockSpec((_BM, h_), lambda b, be: (jnp.minimum(b, NBm - 1), 0)),
            pl.BlockSpec((_BM, h_), lambda b, be: (jnp.maximum(b - NBm, 0), 0)),
            pl.BlockSpec((1, h_, I), lambda b, be: (be[b], 0, 0)),
            pl.BlockSpec((1, h_, I), lambda b, be: (be[b], 0, 0)),
            pl.BlockSpec((1, I, h_), lambda b, be: (be[b], 0, 0)),
            pl.BlockSpec((1, 1, I), lambda b, be: (be[b], 0, 0)),
            pl.BlockSpec((1, 1, I), lambda b, be: (be[b], 0, 0)),
            pl.BlockSpec((1, 1, h_), lambda b, be: (be[b], 0, 0)),
            pl.BlockSpec((_BM, 1), lambda b, be: (b, 0)),
        ],
        out_specs=pl.BlockSpec((_BM, h_), lambda b, be: (b, 0)),
    )

    def ffn_body(be_ref, xs_ref, hs_ref, w0_ref, w1_ref, wo_ref, b0_ref,
                 b1_ref, bo_ref, sw_ref, out_ref):
        bidx = pl.program_id(0)
        e = be_ref[bidx]
        xsel = jnp.where(e == EP - 1, hs_ref[...], xs_ref[...]).astype(bf)
        h0 = jnp.dot(xsel, w0_ref[0], preferred_element_type=jnp.float32) \
            + b0_ref[0]
        h1 = jnp.dot(xsel, w1_ref[0], preferred_element_type=jnp.float32) \
            + b1_ref[0]
        inter = (h0 * jax.nn.sigmoid(h0) * h1).astype(bf)
        out = jnp.dot(inter, wo_ref[0], preferred_element_type=jnp.float32) \
            + bo_ref[0]
        out_ref[...] = out * sw_ref[...]

    out_sorted = pl.pallas_call(
        ffn_body,
        grid_spec=grid_spec,
        out_shape=jax.ShapeDtypeStruct((S_pad, h_), jnp.float32),
    )(bex1, x_sorted, hs2, W0c, W1c, Woc, b0c, b1c, boc, slot_w)

    final = _make_sc3(T, h_, S_moe)(out_sorted, pos.reshape(2 * T))
    return final.reshape(b_, s_, h_), logits


# speedup vs baseline: 1.0008x; 1.0008x over previous
import functools

import jax
import jax.numpy as jnp
from jax.experimental import pallas as pl
from jax.experimental.pallas import tpu as pltpu
from jax.experimental.pallas import tpu_sc as plsc

_BM = 256


def _router_body(hs_ref, gw_ref, logits_ref, pos_ref, w_ref, sg_ref, bex_ref):
    hs = hs_ref[...]
    T = hs.shape[0]
    EP = gw_ref.shape[1]
    E = EP - 1
    NBm = (2 * T) // _BM + E
    NB = NBm + T // _BM
    l9 = jnp.dot(hs, gw_ref[...], preferred_element_type=jnp.float32)
    logits_ref[...] = l9[:, :E]
    lane = jax.lax.broadcasted_iota(jnp.int32, (T, EP), 1)
    moe = lane < E
    lm = jnp.where(moe, l9, -1e30)
    mx = jnp.max(lm, axis=1, keepdims=True)
    ex = jnp.where(moe, jnp.exp(lm - mx), 0.0)
    rw = ex / jnp.sum(ex, axis=1, keepdims=True)
    m1 = jnp.max(rw, axis=1, keepdims=True)
    e0 = jnp.min(jnp.where(rw == m1, lane, EP), axis=1, keepdims=True)
    rw2 = jnp.where(lane == e0, -1.0, rw)
    m2 = jnp.max(rw2, axis=1, keepdims=True)
    e1 = jnp.min(jnp.where(rw2 == m2, lane, EP), axis=1, keepdims=True)
    den = m1 + m2
    w_ref[:, 0:1] = m1 / den
    w_ref[:, 1:2] = m2 / den
    sg_ref[...] = jax.nn.sigmoid(l9[:, E:EP])

    oh0 = (lane[:, :E] == e0).astype(jnp.float32)
    oh1 = (lane[:, :E] == e1).astype(jnp.float32)
    m8 = oh0 + oh1
    r_i = jax.lax.broadcasted_iota(jnp.int32, (T, T), 0)
    c_i = jax.lax.broadcasted_iota(jnp.int32, (T, T), 1)
    ltri = (c_i < r_i).astype(jnp.bfloat16)
    rank = jnp.dot(ltri, m8.astype(jnp.bfloat16),
                   preferred_element_type=jnp.float32)
    ones_row = jnp.ones((1, T), jnp.bfloat16)
    cnt = jnp.dot(ones_row, m8.astype(jnp.bfloat16),
                  preferred_element_type=jnp.float32)
    ci = cnt.astype(jnp.int32)
    pci = ((ci + (_BM - 1)) // _BM) * _BM
    pcf = pci.astype(jnp.float32)
    u_r = jax.lax.broadcasted_iota(jnp.int32, (E, E), 0)
    u_c = jax.lax.broadcasted_iota(jnp.int32, (E, E), 1)
    utri = (u_r < u_c).astype(jnp.float32)
    poff = jnp.dot(pcf, utri, preferred_element_type=jnp.float32)
    pos0 = jnp.sum((rank + poff) * oh0, axis=1, keepdims=True)
    pos1 = jnp.sum((rank + poff) * oh1, axis=1, keepdims=True)
    pos_ref[:, 0:1] = pos0.astype(jnp.int32)
    pos_ref[:, 1:2] = pos1.astype(jnp.int32)

    bio = jax.lax.broadcasted_iota(jnp.int32, (NB, 1), 0)
    bstart = (bio * _BM).astype(jnp.float32)
    ends = poff + pcf
    be8 = jnp.sum((bstart >= ends).astype(jnp.int32), axis=1, keepdims=True)
    bex_ref[...] = jnp.where(bio >= NBm, E, jnp.minimum(be8, E - 1))


def _make_sc1(T, S_moe):
    mesh = plsc.VectorSubcoreMesh(core_axis_name="c", subcore_axis_name="s")

    @functools.partial(
        pl.kernel, mesh=mesh,
        compiler_params=pltpu.CompilerParams(needs_layout_passes=False),
        out_type=[jax.ShapeDtypeStruct((S_moe,), jnp.int32),
                  jax.ShapeDtypeStruct((S_moe,), jnp.float32)],
        scratch_types=[pltpu.VMEM((2 * T,), jnp.int32),
                       pltpu.VMEM((2 * T,), jnp.float32),
                       pltpu.VMEM((S_moe,), jnp.int32),
                       pltpu.VMEM((S_moe,), jnp.float32)])
    def sc1(pos_hbm, w_hbm, tok_out, w_out, pos_v, w_v, stok_v, sw_v):
        cid = jax.lax.axis_index("c")
        sid = jax.lax.axis_index("s")

        @pl.when(jnp.logical_and(cid == 0, sid == 0))
        def _():
            pltpu.sync_copy(pos_hbm, pos_v)
            pltpu.sync_copy(w_hbm, w_v)
            zi = jnp.zeros((16,), jnp.int32)
            zf = jnp.zeros((16,), jnp.float32)

            def zbody(i, carry):
                stok_v[pl.ds(i * 16, 16)] = zi
                sw_v[pl.ds(i * 16, 16)] = zf
                return carry

            jax.lax.fori_loop(0, S_moe // 16, zbody, 0)
            lanes = jax.lax.broadcasted_iota(jnp.int32, (16,), 0)

            def sbody(i, carry):
                base = i * 16
                idx = pos_v[pl.ds(base, 16)]
                tok = jax.lax.shift_right_logical(base + lanes, 1)
                plsc.store_scatter(stok_v, [idx], tok)
                wv = w_v[pl.ds(base, 16)]
                plsc.store_scatter(sw_v, [idx], wv)
                return carry

            jax.lax.fori_loop(0, (2 * T) // 16, sbody, 0)
            pltpu.sync_copy(stok_v, tok_out)
            pltpu.sync_copy(sw_v, w_out)

    return sc1


def _make_sc2(S_moe, H):
    NW = 32
    per = S_moe // NW
    CH = 24
    NCH = per // CH
    mesh = plsc.VectorSubcoreMesh(core_axis_name="c", subcore_axis_name="s")

    @functools.partial(
        pl.kernel, mesh=mesh,
        compiler_params=pltpu.CompilerParams(needs_layout_passes=False),
        out_type=jax.ShapeDtypeStruct((S_moe, H), jnp.float32),
        scratch_types=[pltpu.VMEM((per,), jnp.int32),
                       pltpu.VMEM((CH, H), jnp.float32),
                       pltpu.VMEM((CH, H), jnp.float32),
                       pltpu.SemaphoreType.DMA,
                       pltpu.SemaphoreType.DMA,
                       pltpu.SemaphoreType.DMA,
                       pltpu.SemaphoreType.DMA])
    def sc2(tok_hbm, hs_hbm, x_out, idx_v, b0, b1, g0, g1, w0s, w1s):
        cid = jax.lax.axis_index("c")
        sid = jax.lax.axis_index("s")
        wid = sid * 2 + cid
        base = wid * per
        pltpu.sync_copy(tok_hbm.at[pl.ds(base, per)], idx_v)
        bufs = [b0, b1]
        gsems = [g0, g1]
        wsems = [w0s, w1s]
        gd = [None, None]
        wd = [None, None]
        gd[0] = pltpu.async_copy(hs_hbm.at[idx_v.at[pl.ds(0, CH)]], bufs[0],
                                 gsems[0])
        for c in range(NCH):
            cb = c % 2
            nb = (c + 1) % 2
            if c + 1 < NCH:
                if c >= 1:
                    wd[nb].wait()
                gd[nb] = pltpu.async_copy(
                    hs_hbm.at[idx_v.at[pl.ds((c + 1) * CH, CH)]], bufs[nb],
                    gsems[nb])
            gd[cb].wait()
            wd[cb] = pltpu.async_copy(bufs[cb],
                                      x_out.at[pl.ds(base + c * CH, CH)],
                                      wsems[cb])
        wd[0].wait()
        wd[1].wait()

    return sc2


def _make_sc3(T, H, S_moe):
    NW = 32
    ntok = T // NW
    CT = 16
    NCH = ntok // CT
    mesh = plsc.VectorSubcoreMesh(core_axis_name="c", subcore_axis_name="s")

    @functools.partial(
        pl.kernel, mesh=mesh,
        compiler_params=pltpu.CompilerParams(needs_layout_passes=False),
        out_type=jax.ShapeDtypeStruct((T, H), jnp.float32),
        scratch_types=[pltpu.VMEM((2 * ntok,), jnp.int32),
                       pltpu.VMEM((CT, H), jnp.float32),
                       pltpu.VMEM((CT, H), jnp.float32),
                       pltpu.VMEM((CT, H), jnp.float32),
                       pltpu.VMEM((16,), jnp.int32),
                       pltpu.VMEM((16,), jnp.int32),
                       pltpu.SemaphoreType.DMA,
                       pltpu.SemaphoreType.DMA,
                       pltpu.SemaphoreType.DMA,
                       pltpu.SemaphoreType.DMA])
    def sc3(osort_hbm, pos_hbm, fin_out, pv, b0, b1, b2, i0v, i1v,
            s0, s1, s2, ws):
        cid = jax.lax.axis_index("c")
        sid = jax.lax.axis_index("s")
        wid = sid * 2 + cid
        base_t = wid * ntok
        pltpu.sync_copy(pos_hbm.at[pl.ds(2 * base_t, 2 * ntok)], pv)
        lanes = jax.lax.broadcasted_iota(jnp.int32, (16,), 0)
        for c in range(NCH):
            toff = c * CT
            g = 2 * toff + 2 * lanes
            i0v[...] = plsc.load_gather(pv, [g])
            i1v[...] = plsc.load_gather(pv, [g + 1])
            d0 = pltpu.async_copy(osort_hbm.at[i0v], b0, s0)
            d1 = pltpu.async_copy(osort_hbm.at[i1v], b1, s1)
            d2 = pltpu.async_copy(
                osort_hbm.at[pl.ds(S_moe + base_t + toff, CT)], b2, s2)
            d0.wait()
            d1.wait()
            d2.wait()
            for r in range(CT):
                def abody(j, carry):
                    for q in range(4):
                        sl = pl.ds(j * 64 + q * 16, 16)
                        b2[r, sl] = b0[r, sl] + b1[r, sl] + b2[r, sl]
                    return carry

                jax.lax.fori_loop(0, H // 64, abody, 0)
            dw = pltpu.async_copy(b2, fin_out.at[pl.ds(base_t + toff, CT)], ws)
            dw.wait()

    return sc3


def kernel(hidden_states, gate_w, W0, b0, W1, b1, Wo, bo, sW0, sb0, sW1, sb1,
           sWo, sbo, sg_w):
    b_, s_, h_ = hidden_states.shape
    T = b_ * s_
    E = gate_w.shape[1]
    I = W0.shape[2]
    EP = E + 1
    NBm = (2 * T) // _BM + E
    NBs = T // _BM
    NB = NBm + NBs
    S_moe = NBm * _BM
    S_pad = NB * _BM
    hs2 = hidden_states.reshape(T, h_)
    gwcat = jnp.concatenate([gate_w, sg_w], axis=1)

    logits, pos, w2, sgate, bex = pl.pallas_call(
        _router_body,
        out_shape=[
            jax.ShapeDtypeStruct((T, E), jnp.float32),
            jax.ShapeDtypeStruct((T, 2), jnp.int32),
            jax.ShapeDtypeStruct((T, 2), jnp.float32),
            jax.ShapeDtypeStruct((T, 1), jnp.float32),
            jax.ShapeDtypeStruct((NB, 1), jnp.int32),
        ],
    )(hs2, gwcat)

    slot_tok, slot_w_moe = _make_sc1(T, S_moe)(pos.reshape(2 * T),
                                               w2.reshape(2 * T))
    x_sorted = _make_sc2(S_moe, h_)(slot_tok, hs2)

    bf = jnp.bfloat16
    W0c = jnp.concatenate([W0, sW0[None]], axis=0).astype(bf)
    W1c = jnp.concatenate([W1, sW1[None]], axis=0).astype(bf)
    Woc = jnp.concatenate([Wo, sWo[None]], axis=0).astype(bf)
    b0c = jnp.concatenate([b0, sb0[None]], axis=0).reshape(EP, 1, I)
    b1c = jnp.concatenate([b1, sb1[None]], axis=0).reshape(EP, 1, I)
    boc = jnp.concatenate([bo, sbo[None]], axis=0).reshape(EP, 1, h_)
    slot_w = jnp.concatenate([slot_w_moe, sgate.reshape(T)]).reshape(S_pad, 1)
    bex1 = bex.reshape(NB)

    grid_spec = pltpu.PrefetchScalarGridSpec(
        num_scalar_prefetch=1,
        grid=(NB,),
        in_specs=[
            pl.BlockSpec((_BM, h_), lambda b, be: (jnp.minimum(b, NBm - 1), 0)),
            pl.BlockSpec((_BM, h_), lambda b, be: (jnp.maximum(b - NBm, 0), 0)),
            pl.BlockSpec((1, h_, I), lambda b, be: (be[b], 0, 0)),
            pl.BlockSpec((1, h_, I), lambda b, be: (be[b], 0, 0)),
            pl.BlockSpec((1, I, h_), lambda b, be: (be[b], 0, 0)),
            pl.BlockSpec((1, 1, I), lambda b, be: (be[b], 0, 0)),
            pl.BlockSpec((1, 1, I), lambda b, be: (be[b], 0, 0)),
            pl.BlockSpec((1, 1, h_), lambda b, be: (be[b], 0, 0)),
            pl.BlockSpec((_BM, 1), lambda b, be: (b, 0)),
        ],
        out_specs=pl.BlockSpec((_BM, h_), lambda b, be: (b, 0)),
    )

    def ffn_body(be_ref, xs_ref, hs_ref, w0_ref, w1_ref, wo_ref, b0_ref,
                 b1_ref, bo_ref, sw_ref, out_ref):
        bidx = pl.program_id(0)
        e = be_ref[bidx]
        xsel = jnp.where(e == EP - 1, hs_ref[...], xs_ref[...]).astype(bf)
        h0 = jnp.dot(xsel, w0_ref[0], preferred_element_type=jnp.float32) \
            + b0_ref[0]
        h1 = jnp.dot(xsel, w1_ref[0], preferred_element_type=jnp.float32) \
            + b1_ref[0]
        inter = (h0 * jax.nn.sigmoid(h0) * h1).astype(bf)
        out = jnp.dot(inter, wo_ref[0], preferred_element_type=jnp.float32) \
            + bo_ref[0]
        out_ref[...] = out * sw_ref[...]

    out_sorted = pl.pallas_call(
        ffn_body,
        grid_spec=grid_spec,
        out_shape=jax.ShapeDtypeStruct((S_pad, h_), jnp.float32),
    )(bex1, x_sorted, hs2, W0c, W1c, Woc, b0c, b1c, boc, slot_w)

    final = _make_sc3(T, h_, S_moe)(out_sorted, pos.reshape(2 * T))
    return final.reshape(b_, s_, h_), logits


# speedup vs baseline: 1.0780x; 1.0771x over previous
import functools

import jax
import jax.numpy as jnp
from jax.experimental import pallas as pl
from jax.experimental.pallas import tpu as pltpu
from jax.experimental.pallas import tpu_sc as plsc

_BM = 256


def _router_body(hs_ref, gw_ref, logits_ref, pos_ref, w_ref, sg_ref, bex_ref):
    hs = hs_ref[...]
    T = hs.shape[0]
    EP = gw_ref.shape[1]
    E = EP - 1
    NBm = (2 * T) // _BM + E
    NB = NBm + T // _BM
    l9 = jnp.dot(hs, gw_ref[...], preferred_element_type=jnp.float32)
    logits_ref[...] = l9[:, :E]
    lane = jax.lax.broadcasted_iota(jnp.int32, (T, EP), 1)
    moe = lane < E
    lm = jnp.where(moe, l9, -1e30)
    mx = jnp.max(lm, axis=1, keepdims=True)
    ex = jnp.where(moe, jnp.exp(lm - mx), 0.0)
    rw = ex / jnp.sum(ex, axis=1, keepdims=True)
    m1 = jnp.max(rw, axis=1, keepdims=True)
    e0 = jnp.min(jnp.where(rw == m1, lane, EP), axis=1, keepdims=True)
    rw2 = jnp.where(lane == e0, -1.0, rw)
    m2 = jnp.max(rw2, axis=1, keepdims=True)
    e1 = jnp.min(jnp.where(rw2 == m2, lane, EP), axis=1, keepdims=True)
    den = m1 + m2
    w_ref[:, 0:1] = m1 / den
    w_ref[:, 1:2] = m2 / den
    sg_ref[...] = jax.nn.sigmoid(l9[:, E:EP])

    oh0 = (lane[:, :E] == e0).astype(jnp.float32)
    oh1 = (lane[:, :E] == e1).astype(jnp.float32)
    m8 = oh0 + oh1
    r_i = jax.lax.broadcasted_iota(jnp.int32, (T, T), 0)
    c_i = jax.lax.broadcasted_iota(jnp.int32, (T, T), 1)
    ltri = (c_i < r_i).astype(jnp.bfloat16)
    rank = jnp.dot(ltri, m8.astype(jnp.bfloat16),
                   preferred_element_type=jnp.float32)
    ones_row = jnp.ones((1, T), jnp.bfloat16)
    cnt = jnp.dot(ones_row, m8.astype(jnp.bfloat16),
                  preferred_element_type=jnp.float32)
    ci = cnt.astype(jnp.int32)
    pci = ((ci + (_BM - 1)) // _BM) * _BM
    pcf = pci.astype(jnp.float32)
    u_r = jax.lax.broadcasted_iota(jnp.int32, (E, E), 0)
    u_c = jax.lax.broadcasted_iota(jnp.int32, (E, E), 1)
    utri = (u_r < u_c).astype(jnp.float32)
    poff = jnp.dot(pcf, utri, preferred_element_type=jnp.float32)
    pos0 = jnp.sum((rank + poff) * oh0, axis=1, keepdims=True)
    pos1 = jnp.sum((rank + poff) * oh1, axis=1, keepdims=True)
    pos_ref[:, 0:1] = pos0.astype(jnp.int32)
    pos_ref[:, 1:2] = pos1.astype(jnp.int32)

    bio = jax.lax.broadcasted_iota(jnp.int32, (NB, 1), 0)
    bstart = (bio * _BM).astype(jnp.float32)
    ends = poff + pcf
    be8 = jnp.sum((bstart >= ends).astype(jnp.int32), axis=1, keepdims=True)
    bex_ref[...] = jnp.where(bio >= NBm, E, jnp.minimum(be8, E - 1))


def _make_sc1(T, S_moe):
    mesh = plsc.VectorSubcoreMesh(core_axis_name="c", subcore_axis_name="s")

    @functools.partial(
        pl.kernel, mesh=mesh,
        compiler_params=pltpu.CompilerParams(needs_layout_passes=False),
        out_type=[jax.ShapeDtypeStruct((S_moe,), jnp.int32),
                  jax.ShapeDtypeStruct((S_moe,), jnp.float32)],
        scratch_types=[pltpu.VMEM((2 * T,), jnp.int32),
                       pltpu.VMEM((2 * T,), jnp.float32),
                       pltpu.VMEM((S_moe,), jnp.int32),
                       pltpu.VMEM((S_moe,), jnp.float32)])
    def sc1(pos_hbm, w_hbm, tok_out, w_out, pos_v, w_v, stok_v, sw_v):
        cid = jax.lax.axis_index("c")
        sid = jax.lax.axis_index("s")

        @pl.when(jnp.logical_and(cid == 0, sid == 0))
        def _():
            pltpu.sync_copy(pos_hbm, pos_v)
            pltpu.sync_copy(w_hbm, w_v)
            zi = jnp.zeros((16,), jnp.int32)
            zf = jnp.zeros((16,), jnp.float32)

            def zbody(i, carry):
                stok_v[pl.ds(i * 16, 16)] = zi
                sw_v[pl.ds(i * 16, 16)] = zf
                return carry

            jax.lax.fori_loop(0, S_moe // 16, zbody, 0)
            lanes = jax.lax.broadcasted_iota(jnp.int32, (16,), 0)

            def sbody(i, carry):
                base = i * 16
                idx = pos_v[pl.ds(base, 16)]
                tok = jax.lax.shift_right_logical(base + lanes, 1)
                plsc.store_scatter(stok_v, [idx], tok)
                wv = w_v[pl.ds(base, 16)]
                plsc.store_scatter(sw_v, [idx], wv)
                return carry

            jax.lax.fori_loop(0, (2 * T) // 16, sbody, 0)
            pltpu.sync_copy(stok_v, tok_out)
            pltpu.sync_copy(sw_v, w_out)

    return sc1


def _make_sc2(S_moe, H):
    NW = 32
    per = S_moe // NW
    CH = 24
    NCH = per // CH
    mesh = plsc.VectorSubcoreMesh(core_axis_name="c", subcore_axis_name="s")

    @functools.partial(
        pl.kernel, mesh=mesh,
        compiler_params=pltpu.CompilerParams(needs_layout_passes=False),
        out_type=jax.ShapeDtypeStruct((S_moe, H), jnp.float32),
        scratch_types=[pltpu.VMEM((per,), jnp.int32),
                       pltpu.VMEM((CH, H), jnp.float32),
                       pltpu.VMEM((CH, H), jnp.float32),
                       pltpu.SemaphoreType.DMA,
                       pltpu.SemaphoreType.DMA,
                       pltpu.SemaphoreType.DMA,
                       pltpu.SemaphoreType.DMA])
    def sc2(tok_hbm, hs_hbm, x_out, idx_v, b0, b1, g0, g1, w0s, w1s):
        cid = jax.lax.axis_index("c")
        sid = jax.lax.axis_index("s")
        wid = sid * 2 + cid
        base = wid * per
        pltpu.sync_copy(tok_hbm.at[pl.ds(base, per)], idx_v)
        bufs = [b0, b1]
        gsems = [g0, g1]
        wsems = [w0s, w1s]
        gd = [None, None]
        wd = [None, None]
        gd[0] = pltpu.async_copy(hs_hbm.at[idx_v.at[pl.ds(0, CH)]], bufs[0],
                                 gsems[0])
        for c in range(NCH):
            cb = c % 2
            nb = (c + 1) % 2
            if c + 1 < NCH:
                if c >= 1:
                    wd[nb].wait()
                gd[nb] = pltpu.async_copy(
                    hs_hbm.at[idx_v.at[pl.ds((c + 1) * CH, CH)]], bufs[nb],
                    gsems[nb])
            gd[cb].wait()
            wd[cb] = pltpu.async_copy(bufs[cb],
                                      x_out.at[pl.ds(base + c * CH, CH)],
                                      wsems[cb])
        wd[0].wait()
        wd[1].wait()

    return sc2


def _make_sc3(T, H, S_moe):
    NW = 32
    ntok = T // NW
    CT = 16
    NCH = ntok // CT
    mesh = plsc.VectorSubcoreMesh(core_axis_name="c", subcore_axis_name="s")

    @functools.partial(
        pl.kernel, mesh=mesh,
        compiler_params=pltpu.CompilerParams(needs_layout_passes=False),
        out_type=jax.ShapeDtypeStruct((T, H), jnp.float32),
        scratch_types=[pltpu.VMEM((2 * ntok,), jnp.int32),
                       pltpu.VMEM((CT, H), jnp.float32),
                       pltpu.VMEM((CT, H), jnp.float32),
                       pltpu.VMEM((CT, H), jnp.float32),
                       pltpu.VMEM((16,), jnp.int32),
                       pltpu.VMEM((16,), jnp.int32),
                       pltpu.SemaphoreType.DMA,
                       pltpu.SemaphoreType.DMA,
                       pltpu.SemaphoreType.DMA,
                       pltpu.SemaphoreType.DMA])
    def sc3(osort_hbm, sh_hbm, pos_hbm, fin_out, pv, b0, b1, b2, i0v, i1v,
            s0, s1, s2, ws):
        cid = jax.lax.axis_index("c")
        sid = jax.lax.axis_index("s")
        wid = sid * 2 + cid
        base_t = wid * ntok
        pltpu.sync_copy(pos_hbm.at[pl.ds(2 * base_t, 2 * ntok)], pv)
        lanes = jax.lax.broadcasted_iota(jnp.int32, (16,), 0)
        for c in range(NCH):
            toff = c * CT
            g = 2 * toff + 2 * lanes
            i0v[...] = plsc.load_gather(pv, [g])
            i1v[...] = plsc.load_gather(pv, [g + 1])
            d0 = pltpu.async_copy(osort_hbm.at[i0v], b0, s0)
            d1 = pltpu.async_copy(osort_hbm.at[i1v], b1, s1)
            d2 = pltpu.async_copy(
                sh_hbm.at[pl.ds(base_t + toff, CT)], b2, s2)
            d0.wait()
            d1.wait()
            d2.wait()
            for r in range(CT):
                def abody(j, carry):
                    for q in range(4):
                        sl = pl.ds(j * 64 + q * 16, 16)
                        b2[r, sl] = b0[r, sl] + b1[r, sl] + b2[r, sl]
                    return carry

                jax.lax.fori_loop(0, H // 64, abody, 0)
            dw = pltpu.async_copy(b2, fin_out.at[pl.ds(base_t + toff, CT)], ws)
            dw.wait()

    return sc3


def kernel(hidden_states, gate_w, W0, b0, W1, b1, Wo, bo, sW0, sb0, sW1, sb1,
           sWo, sbo, sg_w):
    b_, s_, h_ = hidden_states.shape
    T = b_ * s_
    E = gate_w.shape[1]
    I = W0.shape[2]
    EP = E + 1
    NBm = (2 * T) // _BM + E
    NBs = T // _BM
    NB = NBm + NBs
    S_moe = NBm * _BM
    S_pad = NB * _BM
    hs2 = hidden_states.reshape(T, h_)
    gwcat = jnp.concatenate([gate_w, sg_w], axis=1)

    logits, pos, w2, sgate, bex = pl.pallas_call(
        _router_body,
        out_shape=[
            jax.ShapeDtypeStruct((T, E), jnp.float32),
            jax.ShapeDtypeStruct((T, 2), jnp.int32),
            jax.ShapeDtypeStruct((T, 2), jnp.float32),
            jax.ShapeDtypeStruct((T, 1), jnp.float32),
            jax.ShapeDtypeStruct((NB, 1), jnp.int32),
        ],
    )(hs2, gwcat)

    slot_tok, slot_w_moe = _make_sc1(T, S_moe)(pos.reshape(2 * T),
                                               w2.reshape(2 * T))
    x_sorted = _make_sc2(S_moe, h_)(slot_tok, hs2)

    bf = jnp.bfloat16
    W0b = W0.astype(bf)
    W1b = W1.astype(bf)
    Wob = Wo.astype(bf)
    b0r = b0.reshape(E, 1, I)
    b1r = b1.reshape(E, 1, I)
    bor = bo.reshape(E, 1, h_)
    slot_w = slot_w_moe.reshape(S_moe, 1)
    bexm = bex.reshape(NB)[:NBm]

    def sh_body(hs_ref, w0_ref, w1_ref, wo_ref, b0_ref, b1_ref, bo_ref,
                sg_ref, out_ref):
        x = hs_ref[...].astype(bf)
        h0 = jnp.dot(x, w0_ref[...], preferred_element_type=jnp.float32) \
            + b0_ref[0]
        h1 = jnp.dot(x, w1_ref[...], preferred_element_type=jnp.float32) \
            + b1_ref[0]
        inter = (h0 * jax.nn.sigmoid(h0) * h1).astype(bf)
        out = jnp.dot(inter, wo_ref[...], preferred_element_type=jnp.float32) \
            + bo_ref[0]
        out_ref[...] = out * sg_ref[...]

    shared_out = pl.pallas_call(
        sh_body,
        grid=(T // _BM,),
        in_specs=[
            pl.BlockSpec((_BM, h_), lambda i: (i, 0)),
            pl.BlockSpec((h_, I), lambda i: (0, 0)),
            pl.BlockSpec((h_, I), lambda i: (0, 0)),
            pl.BlockSpec((I, h_), lambda i: (0, 0)),
            pl.BlockSpec((1, I), lambda i: (0, 0)),
            pl.BlockSpec((1, I), lambda i: (0, 0)),
            pl.BlockSpec((1, h_), lambda i: (0, 0)),
            pl.BlockSpec((_BM, 1), lambda i: (i, 0)),
        ],
        out_specs=pl.BlockSpec((_BM, h_), lambda i: (i, 0)),
        out_shape=jax.ShapeDtypeStruct((T, h_), jnp.float32),
    )(hs2, sW0.astype(bf), sW1.astype(bf), sWo.astype(bf),
      sb0.reshape(1, I), sb1.reshape(1, I), sbo.reshape(1, h_), sgate)

    grid_spec = pltpu.PrefetchScalarGridSpec(
        num_scalar_prefetch=1,
        grid=(NBm,),
        in_specs=[
            pl.BlockSpec((_BM, h_), lambda b, be: (b, 0)),
            pl.BlockSpec((1, h_, I), lambda b, be: (be[b], 0, 0)),
            pl.BlockSpec((1, h_, I), lambda b, be: (be[b], 0, 0)),
            pl.BlockSpec((1, I, h_), lambda b, be: (be[b], 0, 0)),
            pl.BlockSpec((1, 1, I), lambda b, be: (be[b], 0, 0)),
            pl.BlockSpec((1, 1, I), lambda b, be: (be[b], 0, 0)),
            pl.BlockSpec((1, 1, h_), lambda b, be: (be[b], 0, 0)),
            pl.BlockSpec((_BM, 1), lambda b, be: (b, 0)),
        ],
        out_specs=pl.BlockSpec((_BM, h_), lambda b, be: (b, 0)),
    )

    def ffn_body(be_ref, xs_ref, w0_ref, w1_ref, wo_ref, b0_ref,
                 b1_ref, bo_ref, sw_ref, out_ref):
        x = xs_ref[...].astype(bf)
        h0 = jnp.dot(x, w0_ref[0], preferred_element_type=jnp.float32) \
            + b0_ref[0]
        h1 = jnp.dot(x, w1_ref[0], preferred_element_type=jnp.float32) \
            + b1_ref[0]
        inter = (h0 * jax.nn.sigmoid(h0) * h1).astype(bf)
        out = jnp.dot(inter, wo_ref[0], preferred_element_type=jnp.float32) \
            + bo_ref[0]
        out_ref[...] = out * sw_ref[...]

    out_sorted = pl.pallas_call(
        ffn_body,
        grid_spec=grid_spec,
        out_shape=jax.ShapeDtypeStruct((S_moe, h_), jnp.float32),
    )(bexm, x_sorted, W0b, W1b, Wob, b0r, b1r, bor, slot_w)

    final = _make_sc3(T, h_, S_moe)(out_sorted, shared_out,
                                    pos.reshape(2 * T))
    return final.reshape(b_, s_, h_), logits


# speedup vs baseline: 1.6591x; 1.5390x over previous
import functools

import jax
import jax.numpy as jnp
from jax.experimental import pallas as pl
from jax.experimental.pallas import tpu as pltpu
from jax.experimental.pallas import tpu_sc as plsc


def _router_body(hs_ref, gw_ref, logits_ref, l9_ref):
    l9 = jnp.dot(hs_ref[...], gw_ref[...], preferred_element_type=jnp.float32)
    E = gw_ref.shape[1] - 1
    logits_ref[...] = l9[:, :E]
    l9_ref[...] = l9


def _make_sc_gate(T, EP):
    E = EP - 1
    NW = 16
    ntok = T // NW
    mesh = plsc.VectorSubcoreMesh(core_axis_name="c", subcore_axis_name="s")

    @functools.partial(
        pl.kernel, mesh=mesh,
        compiler_params=pltpu.CompilerParams(needs_layout_passes=False),
        out_type=jax.ShapeDtypeStruct((EP, T), jnp.float32),
        scratch_types=[pltpu.VMEM((EP, ntok), jnp.float32),
                       pltpu.VMEM((EP, ntok), jnp.float32)])
    def sc_gate(l9t_hbm, dwt_out, lv, dv):
        cid = jax.lax.axis_index("c")
        sid = jax.lax.axis_index("s")
        wid = sid * 2 + cid

        @pl.when(wid < NW)
        def _():
            _gate_work(l9t_hbm, dwt_out, lv, dv, wid)

    def _gate_work(l9t_hbm, dwt_out, lv, dv, wid):
        base = wid * ntok
        pltpu.sync_copy(l9t_hbm.at[:, pl.ds(base, ntok)], lv)
        for c in range(ntok // 16):
            sl = pl.ds(c * 16, 16)
            v = [lv[e, sl] for e in range(EP)]
            mx = v[0]
            for e in range(1, E):
                mx = jnp.maximum(mx, v[e])
            ex = [jnp.exp(v[e] - mx) for e in range(E)]
            s = ex[0]
            for e in range(1, E):
                s = s + ex[e]
            rw = [ex[e] / s for e in range(E)]
            m1 = rw[0]
            for e in range(1, E):
                m1 = jnp.maximum(m1, rw[e])
            found = jnp.zeros((16,), jnp.bool_)
            sel0 = []
            for e in range(E):
                hit = jnp.logical_and(rw[e] == m1, jnp.logical_not(found))
                sel0.append(hit)
                found = jnp.logical_or(found, hit)
            r2 = [jnp.where(sel0[e], -1.0, rw[e]) for e in range(E)]
            m2 = r2[0]
            for e in range(1, E):
                m2 = jnp.maximum(m2, r2[e])
            found2 = jnp.zeros((16,), jnp.bool_)
            sel1 = []
            for e in range(E):
                hit = jnp.logical_and(r2[e] == m2, jnp.logical_not(found2))
                sel1.append(hit)
                found2 = jnp.logical_or(found2, hit)
            den = m1 + m2
            w0 = m1 / den
            w1 = m2 / den
            zero = jnp.zeros((16,), jnp.float32)
            for e in range(E):
                dv[e, sl] = jnp.where(sel0[e], w0,
                                      jnp.where(sel1[e], w1, zero))
            g = v[E]
            dv[E, sl] = 1.0 / (1.0 + jnp.exp(-g))
        pltpu.sync_copy(dv, dwt_out.at[:, pl.ds(base, ntok)])

    return sc_gate


def _ffn_body(hs_ref, w0_ref, w1_ref, wo_ref, b0_ref, b1_ref, bo_ref, dw_ref,
              out_ref):
    e = pl.program_id(1)
    x = hs_ref[...].astype(jnp.bfloat16)
    h0 = jnp.dot(x, w0_ref[0], preferred_element_type=jnp.float32) + b0_ref[0]
    h1 = jnp.dot(x, w1_ref[0], preferred_element_type=jnp.float32) + b1_ref[0]
    inter = (h0 * jax.nn.sigmoid(h0) * h1).astype(jnp.bfloat16)
    out = jnp.dot(inter, wo_ref[0], preferred_element_type=jnp.float32) + bo_ref[0]
    lane = jax.lax.broadcasted_iota(jnp.int32, dw_ref.shape, 1)
    wcol = jnp.sum(jnp.where(lane == e, dw_ref[...], 0.0), axis=1, keepdims=True)
    contrib = out * wcol

    @pl.when(e == 0)
    def _():
        out_ref[...] = contrib

    @pl.when(e > 0)
    def _():
        out_ref[...] += contrib


def kernel(hidden_states, gate_w, W0, b0, W1, b1, Wo, bo, sW0, sb0, sW1, sb1,
           sWo, sbo, sg_w):
    b_, s_, h_ = hidden_states.shape
    T = b_ * s_
    E = gate_w.shape[1]
    I = W0.shape[2]
    EP = E + 1
    hs2 = hidden_states.reshape(T, h_)
    gwcat = jnp.concatenate([gate_w, sg_w], axis=1)

    logits, l9 = pl.pallas_call(
        _router_body,
        out_shape=[
            jax.ShapeDtypeStruct((T, E), jnp.float32),
            jax.ShapeDtypeStruct((T, EP), jnp.float32),
        ],
    )(hs2, gwcat)

    dwt = _make_sc_gate(T, EP)(l9.T)
    dw = dwt.T

    bf = jnp.bfloat16
    W0c = jnp.concatenate([W0, sW0[None]], axis=0).astype(bf)
    W1c = jnp.concatenate([W1, sW1[None]], axis=0).astype(bf)
    Woc = jnp.concatenate([Wo, sWo[None]], axis=0).astype(bf)
    b0c = jnp.concatenate([b0, sb0[None]], axis=0).reshape(EP, 1, I)
    b1c = jnp.concatenate([b1, sb1[None]], axis=0).reshape(EP, 1, I)
    boc = jnp.concatenate([bo, sbo[None]], axis=0).reshape(EP, 1, h_)

    Bm = 512
    grid = (T // Bm, EP)
    final = pl.pallas_call(
        _ffn_body,
        grid=grid,
        in_specs=[
            pl.BlockSpec((Bm, h_), lambda i, e: (i, 0)),
            pl.BlockSpec((1, h_, I), lambda i, e: (e, 0, 0)),
            pl.BlockSpec((1, h_, I), lambda i, e: (e, 0, 0)),
            pl.BlockSpec((1, I, h_), lambda i, e: (e, 0, 0)),
            pl.BlockSpec((1, 1, I), lambda i, e: (e, 0, 0)),
            pl.BlockSpec((1, 1, I), lambda i, e: (e, 0, 0)),
            pl.BlockSpec((1, 1, h_), lambda i, e: (e, 0, 0)),
            pl.BlockSpec((Bm, EP), lambda i, e: (i, 0)),
        ],
        out_specs=pl.BlockSpec((Bm, h_), lambda i, e: (i, 0)),
        out_shape=jax.ShapeDtypeStruct((T, h_), jnp.float32),
    )(hs2, W0c, W1c, Woc, b0c, b1c, boc, dw)

    return final.reshape(b_, s_, h_), logits
